# baseline (speedup 1.0000x reference)
import functools

import jax
import jax.numpy as jnp
from jax import lax
from jax.experimental import pallas as pl
from jax.experimental.pallas import tpu as pltpu
from jax.experimental.pallas import tpu_sc as plsc

_SIZE = 16384
_SMOOTH = 0.1
_CONF = 0.9
_S = _SMOOTH / (_SIZE - 2)
_LANE_BITS = 7
_LANE_MASK = 127


def _sc_gather(xf_hbm, t_hbm, out_hbm, idx_v, gidx_v, elems_v, acc_v, sem,
               *, bpw, lanes, ncores, vocab):
    wid = lax.axis_index("s") * ncores + lax.axis_index("c")
    base = wid * bpw
    pltpu.sync_copy(t_hbm.at[pl.ds(base, bpw)], idx_v)
    for k in range(bpw // lanes):
        t16 = idx_v[pl.ds(k * lanes, lanes)]
        lane = lax.iota(jnp.int32, lanes)
        g = (base + k * lanes + lane) * vocab + t16
        gidx_v[pl.ds(k * lanes, lanes)] = g
    pltpu.async_copy(xf_hbm.at[gidx_v], elems_v, sem).wait()
    acc = jnp.zeros((lanes,), jnp.float32)
    for k in range(bpw // lanes):
        t16 = idx_v[pl.ds(k * lanes, lanes)]
        val = elems_v[pl.ds(k * lanes, lanes)]
        acc = acc + jnp.where(t16 != 0, val, 0.0)
    acc_v[...] = acc
    pltpu.sync_copy(acc_v, out_hbm.at[wid])


def _target_partials(x, target):
    info = plsc.get_sparse_core_info()
    nc, ns, lanes = info.num_cores, info.num_subcores, info.num_lanes
    nw = nc * ns
    n, vocab = x.shape
    bpw = n // nw
    xf = x.reshape(-1)
    mesh = plsc.VectorSubcoreMesh(core_axis_name="c", subcore_axis_name="s")
    return pl.kernel(
        functools.partial(_sc_gather, bpw=bpw, lanes=lanes, ncores=nc,
                          vocab=vocab),
        out_type=jax.ShapeDtypeStruct((nw, lanes), jnp.float32),
        mesh=mesh,
        scratch_types=[
            pltpu.VMEM((bpw,), jnp.int32),
            pltpu.VMEM((bpw,), jnp.int32),
            pltpu.VMEM((bpw,), jnp.float32),
            pltpu.VMEM((lanes,), jnp.float32),
            pltpu.SemaphoreType.DMA,
        ],
    )(xf, target)


def _tc_kernel(t_ref, p_ref, x_ref, o_ref, *, n_blocks):
    i = pl.program_id(0)
    xb = x_ref[...]
    tcol = t_ref[0]
    rowsum = jnp.sum(xb, axis=1, keepdims=True)
    x0 = xb[:, 0:1]
    k_const = _SMOOTH * jnp.log(_S) + _CONF * jnp.log(_CONF)
    contrib = jnp.where(tcol != 0, k_const - _S * rowsum + _S * x0, 0.0)
    total = jnp.sum(contrib).reshape(1, 1)

    @pl.when(i == 0)
    def _():
        o_ref[...] = jnp.zeros_like(o_ref)

    o_ref[...] += total

    @pl.when(i == n_blocks - 1)
    def _():
        o_ref[...] += ((_S - _CONF) * jnp.sum(p_ref[0])).reshape(1, 1)


def kernel(x, target):
    n, c = x.shape
    br = 256
    n_blocks = n // br
    partials = _target_partials(x, target)
    tr = target.reshape(n_blocks, br, 1)
    pr = partials.reshape(1, *partials.shape)
    out = pl.pallas_call(
        functools.partial(_tc_kernel, n_blocks=n_blocks),
        grid=(n_blocks,),
        in_specs=[
            pl.BlockSpec((1, br, 1), lambda i: (i, 0, 0)),
            pl.BlockSpec((1,) + partials.shape, lambda i: (0, 0, 0)),
            pl.BlockSpec((br, c), lambda i: (i, 0)),
        ],
        out_specs=pl.BlockSpec((1, 1), lambda i: (0, 0)),
        out_shape=jax.ShapeDtypeStruct((1, 1), jnp.float32),
    )(tr, pr, x)
    return out[0, 0]


# speedup vs baseline: 3.2716x; 3.2716x over previous
import functools

import jax
import jax.numpy as jnp
from jax.experimental import pallas as pl
from jax.experimental.pallas import tpu as pltpu

_SIZE = 16384
_SMOOTH = 0.1
_CONF = 0.9
_S = _SMOOTH / (_SIZE - 2)


def _ls_kernel(ts_ref, t_ref, x_ref, o_ref, *, n_blocks):
    i = pl.program_id(0)
    xb = x_ref[...]
    tcol = t_ref[0]
    br, c = xb.shape
    rowsum = jnp.sum(xb, axis=1, keepdims=True)
    x0 = xb[:, 0:1]
    k_const = _SMOOTH * jnp.log(_S) + _CONF * jnp.log(_CONF)
    contrib = jnp.where(tcol != 0, k_const - _S * rowsum + _S * x0, 0.0)

    lane = jax.lax.broadcasted_iota(jnp.int32, (1, 128), 1)
    acc = jnp.zeros((1, 128), jnp.float32)
    for r in range(br):
        t = ts_ref[0, r, 0]
        v = x_ref[pl.ds(r, 1), pl.ds((t >> 7) * 128, 128)]
        acc += jnp.where((lane == (t & 127)) & (t != 0), v, 0.0)

    total = (jnp.sum(contrib) + (_S - _CONF) * jnp.sum(acc)).reshape(1, 1)

    @pl.when(i == 0)
    def _():
        o_ref[...] = jnp.zeros_like(o_ref)

    o_ref[...] += total


def kernel(x, target):
    n, c = x.shape
    br = 256
    n_blocks = n // br
    tr = target.reshape(n_blocks, br, 1)
    out = pl.pallas_call(
        functools.partial(_ls_kernel, n_blocks=n_blocks),
        grid=(n_blocks,),
        in_specs=[
            pl.BlockSpec((1, br, 1), lambda i: (i, 0, 0),
                         memory_space=pltpu.SMEM),
            pl.BlockSpec((1, br, 1), lambda i: (i, 0, 0)),
            pl.BlockSpec((br, c), lambda i: (i, 0)),
        ],
        out_specs=pl.BlockSpec((1, 1), lambda i: (0, 0)),
        out_shape=jax.ShapeDtypeStruct((1, 1), jnp.float32),
    )(tr, tr, x)
    return out[0, 0]


# speedup vs baseline: 3.4281x; 1.0478x over previous
import functools

import jax
import jax.numpy as jnp
from jax.experimental import pallas as pl
from jax.experimental.pallas import tpu as pltpu

_SIZE = 16384
_SMOOTH = 0.1
_CONF = 0.9
_S = _SMOOTH / (_SIZE - 2)


def _ls_kernel(ts_ref, t_ref, x_ref, o_ref, *, n_blocks):
    i = pl.program_id(0)
    xb = x_ref[...]
    tcol = t_ref[0]
    br, c = xb.shape
    rowsum = jnp.sum(xb, axis=1, keepdims=True)
    x0 = xb[:, 0:1]
    k_const = _SMOOTH * jnp.log(_S) + _CONF * jnp.log(_CONF)
    contrib = jnp.where(tcol != 0, k_const - _S * rowsum + _S * x0, 0.0)

    lane = jax.lax.broadcasted_iota(jnp.int32, (1, 128), 1)
    acc = jnp.zeros((1, 128), jnp.float32)
    for r in range(br):
        t = ts_ref[0, r, 0]
        v = x_ref[pl.ds(r, 1), pl.ds((t >> 7) * 128, 128)]
        acc += jnp.where((lane == (t & 127)) & (t != 0), v, 0.0)

    total = (jnp.sum(contrib) + (_S - _CONF) * jnp.sum(acc)).reshape(1, 1)

    @pl.when(i == 0)
    def _():
        o_ref[...] = jnp.zeros_like(o_ref)

    o_ref[...] += total


def kernel(x, target):
    n, c = x.shape
    br = 128
    n_blocks = n // br
    tr = target.reshape(n_blocks, br, 1)
    out = pl.pallas_call(
        functools.partial(_ls_kernel, n_blocks=n_blocks),
        grid=(n_blocks,),
        in_specs=[
            pl.BlockSpec((1, br, 1), lambda i: (i, 0, 0),
                         memory_space=pltpu.SMEM),
            pl.BlockSpec((1, br, 1), lambda i: (i, 0, 0)),
            pl.BlockSpec((br, c), lambda i: (i, 0)),
        ],
        out_specs=pl.BlockSpec((1, 1), lambda i: (0, 0)),
        out_shape=jax.ShapeDtypeStruct((1, 1), jnp.float32),
    )(tr, tr, x)
    return out[0, 0]
